# Initial kernel scaffold; baseline (speedup 1.0000x reference)
import jax
import jax.numpy as jnp
from jax.experimental import pallas as pl


def kernel(feat, edge_index, W_conv, b_conv, W0, b0, W1, b1, W2, b2):
    raise NotImplementedError("write your pallas kernel here")


# speedup vs baseline: 29.1427x; 29.1427x over previous
import functools

import jax
import jax.numpy as jnp
from jax import lax
from jax.experimental import pallas as pl
from jax.experimental.pallas import tpu as pltpu
from jax.experimental.pallas import tpu_sc as plsc

B = 2
N = 100000
E = 1600000
D_IN = 48
D_H = 48
D_L0 = 96
D_OUT = 24
DF = 96

NC = 2
NS = 16
NPH = 100352
ER = 12800
EPAD = ER * 128 - E
CH = 12288
NCHUNK = 9
AGGR = NCHUNK * CH
CH2 = CH + 128
DEG_SH = NPH // NS
BRD = 128
EROWS_MSG = ER // NS
BR = 80
QCAP = 256
NBLK = 196
BN = 512

_mesh = plsc.VectorSubcoreMesh(core_axis_name="c", subcore_axis_name="s")


def _fill_const(ref, rows, width, val, dtype):
    v = jnp.full((16,), val, dtype)
    per = width // 16

    def body(i, _):
        ref[i // per, pl.ds((i % per) * 16, 16)] = v
        return _

    lax.fori_loop(0, rows * per, body, None)


def _deg_body(srcp, dstp, out_hbm, sbuf, dbuf, tab_o, tab_i):
    c = lax.axis_index("c")
    s = lax.axis_index("s")
    zv = jnp.zeros((16,), jnp.float32)
    ones_v = jnp.full((16,), 1.0, jnp.float32)
    nbase = s * DEG_SH

    def zloop(k, _):
        tab_o[pl.ds(k * 16, 16)] = zv
        tab_i[pl.ds(k * 16, 16)] = zv
        return _

    lax.fori_loop(0, DEG_SH // 16, zloop, None)

    def bloop(b, _):
        row0 = c * (ER // NC) + b * BRD
        pltpu.sync_copy(srcp.at[pl.ds(row0, BRD)], sbuf)
        pltpu.sync_copy(dstp.at[pl.ds(row0, BRD)], dbuf)

        def row(j, _):
            def grp(g, _):
                sv = sbuf[j, pl.ds(g * 16, 16)]
                lo = sv - nbase
                mo = (lo >= 0) & (lo < DEG_SH)
                plsc.addupdate_scatter(tab_o, [lo], ones_v, mask=mo)
                dv = dbuf[j, pl.ds(g * 16, 16)]
                li = dv - nbase
                mi = (li >= 0) & (li < DEG_SH)
                plsc.addupdate_scatter(tab_i, [li], ones_v, mask=mi)
                return _

            return lax.fori_loop(0, 8, grp, None)

        return lax.fori_loop(0, BRD, row, None)

    lax.fori_loop(0, (ER // NC) // BRD, bloop, None)
    pltpu.sync_copy(tab_o, out_hbm.at[c, 0, pl.ds(nbase, DEG_SH)])
    pltpu.sync_copy(tab_i, out_hbm.at[c, 1, pl.ds(nbase, DEG_SH)])


def _msg_body(srcp, dstp, h_hbm, agg_hbm,
              sbuf, dbuf, qs, qd, sibuf, dibuf, rows_v, zb, agg_sp, gsem):
    c = lax.axis_index("c")
    s = lax.axis_index("s")
    _fill_const(zb, 128, DF, 0.0, jnp.float32)

    def do_flush(cnt):
        for i in range(8):
            sibuf[pl.ds(i * 16, 16)] = qs[pl.ds(i * 16, 16)]
            dibuf[pl.ds(i * 16, 16)] = qd[pl.ds(i * 16, 16)]
        pltpu.async_copy(h_hbm.at[sibuf], rows_v, gsem).wait()
        pltpu.sync_copy(rows_v, agg_sp.at[dibuf], add=True)
        qs[pl.ds(0, 16)] = qs[pl.ds(128, 16)]
        qd[pl.ds(0, 16)] = qd[pl.ds(128, 16)]
        return cnt - 128

    def chunk_body(c7, _):
        @pl.when(lax.rem(c7, 2) == c)
        def _run():
            base = c7 * CH
            for k in range(CH // (128 * NS)):
                pltpu.sync_copy(zb, agg_sp.at[pl.ds((s * (CH // (128 * NS)) + k) * 128, 128)])
            plsc.subcore_barrier()

            def stage(bk, cnt):
                row0 = s * EROWS_MSG + bk * BR
                pltpu.sync_copy(srcp.at[pl.ds(row0, BR)], sbuf)
                pltpu.sync_copy(dstp.at[pl.ds(row0, BR)], dbuf)

                def row(j, cnt):
                    def grp(g, cnt):
                        sv = sbuf[j, pl.ds(g * 16, 16)]
                        dv = dbuf[j, pl.ds(g * 16, 16)]
                        m = (dv >= base) & (dv < base + CH)
                        dl = dv - base
                        plsc.store_compressed(qs.at[pl.ds(cnt, 16)], sv, mask=m)
                        plsc.store_compressed(qd.at[pl.ds(cnt, 16)], dl, mask=m)
                        cnt = cnt + jnp.sum(m.astype(jnp.int32))
                        return lax.cond(cnt >= 128, do_flush, lambda x: x, cnt)

                    return lax.fori_loop(0, 8, grp, cnt)

                return lax.fori_loop(0, BR, row, cnt)

            cnt = lax.fori_loop(0, EROWS_MSG // BR, stage, jnp.int32(0))
            pad_s = jnp.zeros((16,), jnp.int32)
            pad_d = jnp.full((16,), CH, jnp.int32)
            for i in range(8):
                qs[pl.ds(cnt + i * 16, 16)] = pad_s
                qd[pl.ds(cnt + i * 16, 16)] = pad_d
            do_flush(cnt)
            plsc.subcore_barrier()
            for k in range(CH // (128 * NS)):
                r = s * (CH // NS) + k * 128
                pltpu.sync_copy(agg_sp.at[pl.ds(r, 128)],
                                agg_hbm.at[pl.ds(base + r, 128)])
            plsc.subcore_barrier()

        return _

    lax.fori_loop(0, NCHUNK, chunk_body, None)


_deg_call = functools.partial(
    pl.kernel, _deg_body,
    out_type=jax.ShapeDtypeStruct((NC, 2, NPH), jnp.float32),
    mesh=_mesh,
    compiler_params=pltpu.CompilerParams(needs_layout_passes=False),
    scratch_types=[
        pltpu.VMEM((BRD, 128), jnp.int32),
        pltpu.VMEM((BRD, 128), jnp.int32),
        pltpu.VMEM((DEG_SH,), jnp.float32),
        pltpu.VMEM((DEG_SH,), jnp.float32),
    ],
)()

_msg_call = functools.partial(
    pl.kernel, _msg_body,
    out_type=jax.ShapeDtypeStruct((AGGR, DF), jnp.float32),
    mesh=_mesh,
    compiler_params=pltpu.CompilerParams(
        needs_layout_passes=False, use_tc_tiling_on_sc=False),
    scratch_types=[
        pltpu.VMEM((BR, 128), jnp.int32),
        pltpu.VMEM((BR, 128), jnp.int32),
        pltpu.VMEM((QCAP,), jnp.int32),
        pltpu.VMEM((QCAP,), jnp.int32),
        pltpu.VMEM((128,), jnp.int32),
        pltpu.VMEM((128,), jnp.int32),
        pltpu.VMEM((128, DF), jnp.float32),
        pltpu.VMEM((128, DF), jnp.float32),
        pltpu.VMEM_SHARED((CH2, DF), jnp.float32),
        pltpu.SemaphoreType.DMA,
    ],
)()


def _prep_body(feat_ref, do0_ref, do1_ref, di0_ref, di1_ref, h_ref, nd_ref):
    x0 = feat_ref[0]
    x1 = feat_ref[1]
    do = do0_ref[...] + do1_ref[...]
    di = di0_ref[...] + di1_ref[...]
    ns = jnp.where(do > 0, lax.rsqrt(jnp.maximum(do, 1.0)), 0.0)
    nd = jnp.where(di > 0, lax.rsqrt(jnp.maximum(di, 1.0)), 0.0)
    h_ref[:, 0:D_IN] = x0 * ns[:, None]
    h_ref[:, D_IN:DF] = x1 * ns[:, None]
    nd_ref[...] = nd


def _read_body(agg_ref, nd_ref, wc, bc, w0, b0, w1, b1, w2, b2, out_ref):
    nd = nd_ref[...][:, None]

    def mlp(a):
        y = jnp.maximum(jnp.dot(a, wc[...], preferred_element_type=jnp.float32) + bc[...], 0.0)
        y = jnp.maximum(jnp.dot(y, w0[...], preferred_element_type=jnp.float32) + b0[...], 0.0)
        y = jnp.maximum(jnp.dot(y, w1[...], preferred_element_type=jnp.float32) + b1[...], 0.0)
        return jnp.dot(y, w2[...], preferred_element_type=jnp.float32) + b2[...]

    out_ref[0] = mlp(agg_ref[:, 0:D_IN] * nd)
    out_ref[1] = mlp(agg_ref[:, D_IN:DF] * nd)


def _full(shape):
    return pl.BlockSpec(shape, lambda i: tuple(0 for _ in shape))


_prep_call = pl.pallas_call(
    _prep_body,
    grid=(NBLK,),
    in_specs=[
        pl.BlockSpec((B, BN, D_IN), lambda i: (0, i, 0)),
        pl.BlockSpec((BN,), lambda i: (i,)),
        pl.BlockSpec((BN,), lambda i: (i,)),
        pl.BlockSpec((BN,), lambda i: (i,)),
        pl.BlockSpec((BN,), lambda i: (i,)),
    ],
    out_specs=[
        pl.BlockSpec((BN, DF), lambda i: (i, 0)),
        pl.BlockSpec((BN,), lambda i: (i,)),
    ],
    out_shape=[
        jax.ShapeDtypeStruct((NPH, DF), jnp.float32),
        jax.ShapeDtypeStruct((NPH,), jnp.float32),
    ],
)

_read_call = pl.pallas_call(
    _read_body,
    grid=(NBLK,),
    in_specs=[
        pl.BlockSpec((BN, DF), lambda i: (i, 0)),
        pl.BlockSpec((BN,), lambda i: (i,)),
        _full((D_IN, D_H)),
        _full((D_H,)),
        _full((D_H, D_L0)),
        _full((D_L0,)),
        _full((D_L0, D_H)),
        _full((D_H,)),
        _full((D_H, D_OUT)),
        _full((D_OUT,)),
    ],
    out_specs=pl.BlockSpec((B, BN, D_OUT), lambda i: (0, i, 0)),
    out_shape=jax.ShapeDtypeStruct((B, NPH, D_OUT), jnp.float32),
)


def kernel(feat, edge_index, W_conv, b_conv, W0, b0, W1, b1, W2, b2):
    src = edge_index[0]
    dst = edge_index[1]
    pad = jnp.full((EPAD,), N, jnp.int32)
    srcp = jnp.concatenate([src, pad]).reshape(ER, 128)
    dstp = jnp.concatenate([dst, pad]).reshape(ER, 128)
    xf = feat.reshape(B, N, D_IN)
    featp = jnp.pad(xf, ((0, 0), (0, NPH - N), (0, 0)))

    degs = _deg_call(srcp, dstp)
    h, nd = _prep_call(featp, degs[0, 0], degs[1, 0], degs[0, 1], degs[1, 1])
    agg = _msg_call(srcp, dstp, h)
    out = _read_call(agg, nd, W_conv, b_conv, W0, b0, W1, b1, W2, b2)
    return out[:, :N, :].reshape(B, N, 6, 4)


# speedup vs baseline: 71.4680x; 2.4523x over previous
import functools

import jax
import jax.numpy as jnp
from jax import lax
from jax.experimental import pallas as pl
from jax.experimental.pallas import tpu as pltpu
from jax.experimental.pallas import tpu_sc as plsc

B = 2
N = 100000
E = 1600000
D_IN = 48
D_H = 48
D_L0 = 96
D_OUT = 24
DF = 96

NC = 2
NS = 16
NPH = 100352
ER = 12544
EPAD = ER * 128 - E
CH = 12288
NCHUNK = 9
AGGR = NCHUNK * CH
CH2 = CH + 64
EGRP = 8
NRNG = 4
DEG_SH = NPH // NRNG
ROWS_EG = ER // EGRP
BRD = 112
EROWS_MSG = ER // NS
BR = 56
QCAP = 256
NBLK = 196
BN = 512

_mesh = plsc.VectorSubcoreMesh(core_axis_name="c", subcore_axis_name="s")
_sc_params = pltpu.CompilerParams(
    needs_layout_passes=False, use_tc_tiling_on_sc=False)


def _deg_body(srcp, dstp, out_hbm, sbuf, dbuf, tab_o, tab_i):
    c = lax.axis_index("c")
    s = lax.axis_index("s")
    t = c * NS + s
    eg = lax.rem(t, EGRP)
    nbase = (t // EGRP) * DEG_SH
    zv = jnp.zeros((16,), jnp.float32)
    ones_v = jnp.full((16,), 1.0, jnp.float32)

    def zloop(k, _):
        tab_o[pl.ds(k * 16, 16)] = zv
        tab_i[pl.ds(k * 16, 16)] = zv
        return _

    lax.fori_loop(0, DEG_SH // 16, zloop, None)

    def bloop(b, _):
        row0 = eg * ROWS_EG + b * BRD
        pltpu.sync_copy(srcp.at[pl.ds(row0, BRD)], sbuf)
        pltpu.sync_copy(dstp.at[pl.ds(row0, BRD)], dbuf)

        def row(j, _):
            def grp(g, _):
                sv = sbuf[j, pl.ds(g * 16, 16)]
                lo = sv - nbase
                mo = (lo >= 0) & (lo < DEG_SH)
                plsc.addupdate_scatter(tab_o, [lo], ones_v, mask=mo)
                dv = dbuf[j, pl.ds(g * 16, 16)]
                li = dv - nbase
                mi = (li >= 0) & (li < DEG_SH)
                plsc.addupdate_scatter(tab_i, [li], ones_v, mask=mi)
                return _

            return lax.fori_loop(0, 8, grp, None)

        return lax.fori_loop(0, BRD, row, None)

    lax.fori_loop(0, ROWS_EG // BRD, bloop, None)
    pltpu.sync_copy(tab_o, out_hbm.at[eg, 0, pl.ds(nbase, DEG_SH)])
    pltpu.sync_copy(tab_i, out_hbm.at[eg, 1, pl.ds(nbase, DEG_SH)])


def _msg_body(srcp, dstp, h_hbm, agg_hbm,
              sbuf, dbuf, qs, qd, sia, sib, dia, dib, ra, rb, zb, agg_sp,
              gsa, gsb, ssa, ssb):
    c = lax.axis_index("c")
    s = lax.axis_index("s")
    zv = jnp.zeros((16,), jnp.float32)

    def zb_init(i, _):
        zb[i // 6, pl.ds(lax.rem(i, 6) * 16, 16)] = zv
        return _

    lax.fori_loop(0, 64 * 6, zb_init, None)

    sibufs = (sia, sib)
    dibufs = (dia, dib)
    rows = (ra, rb)
    gsems = (gsa, gsb)
    ssems = (ssa, ssb)

    def flush_slot(p, fl, cnt):
        o = 1 - p

        @pl.when(fl >= 1)
        def _():
            pltpu.make_async_copy(h_hbm.at[sibufs[o]], rows[o],
                                  gsems[o]).wait()
            pltpu.async_copy(rows[o], agg_sp.at[dibufs[o]], ssems[o],
                             add=True)

        @pl.when(fl >= 2)
        def _():
            pltpu.make_async_copy(rows[p], agg_sp.at[dibufs[p]],
                                  ssems[p]).wait()

        for i in range(8):
            sibufs[p][pl.ds(i * 16, 16)] = qs[pl.ds(i * 16, 16)]
            dibufs[p][pl.ds(i * 16, 16)] = qd[pl.ds(i * 16, 16)]
        pltpu.async_copy(h_hbm.at[sibufs[p]], rows[p], gsems[p])
        qs[pl.ds(0, 16)] = qs[pl.ds(128, 16)]
        qd[pl.ds(0, 16)] = qd[pl.ds(128, 16)]
        return cnt - 128, fl + 1

    def do_flush(state):
        cnt, fl = state
        return lax.cond(lax.rem(fl, 2) == 0,
                        lambda a, b: flush_slot(0, a, b),
                        lambda a, b: flush_slot(1, a, b),
                        fl, cnt)

    def chunk_body(c7, _):
        @pl.when(lax.rem(c7, 2) == c)
        def _run():
            base = c7 * CH
            zn = CH // (64 * NS)
            for k in range(zn):
                pltpu.sync_copy(zb, agg_sp.at[pl.ds((s * zn + k) * 64, 64)])
            plsc.subcore_barrier()

            def stage(bk, state):
                row0 = s * EROWS_MSG + bk * BR
                pltpu.sync_copy(srcp.at[pl.ds(row0, BR)], sbuf)
                pltpu.sync_copy(dstp.at[pl.ds(row0, BR)], dbuf)

                def row(j, state):
                    def grp(g, state):
                        cnt, fl = state
                        sv = sbuf[j, pl.ds(g * 16, 16)]
                        dv = dbuf[j, pl.ds(g * 16, 16)]
                        m = (dv >= base) & (dv < base + CH)
                        dl = dv - base
                        plsc.store_compressed(qs.at[pl.ds(cnt, 16)], sv,
                                              mask=m)
                        plsc.store_compressed(qd.at[pl.ds(cnt, 16)], dl,
                                              mask=m)
                        cnt = cnt + jnp.sum(m.astype(jnp.int32))
                        return lax.cond(cnt >= 128, do_flush,
                                        lambda x: x, (cnt, fl))

                    return lax.fori_loop(0, 8, grp, state)

                return lax.fori_loop(0, BR, row, state)

            state = lax.fori_loop(0, EROWS_MSG // BR, stage,
                                  (jnp.int32(0), jnp.int32(0)))
            cnt, fl = state
            pad_s = jnp.zeros((16,), jnp.int32)
            pad_d = jnp.full((16,), CH, jnp.int32)
            for i in range(8):
                qs[pl.ds(cnt + i * 16, 16)] = pad_s
                qd[pl.ds(cnt + i * 16, 16)] = pad_d
            _, fl = do_flush((cnt, fl))

            def drain_last(p):
                pltpu.make_async_copy(h_hbm.at[sibufs[p]], rows[p],
                                      gsems[p]).wait()
                pltpu.async_copy(rows[p], agg_sp.at[dibufs[p]], ssems[p],
                                 add=True)
                pltpu.make_async_copy(rows[p], agg_sp.at[dibufs[p]],
                                      ssems[p]).wait()

            lax.cond(lax.rem(fl, 2) == 1,
                     lambda: drain_last(0), lambda: drain_last(1))

            def drain_prev(p):
                pltpu.make_async_copy(rows[p], agg_sp.at[dibufs[p]],
                                      ssems[p]).wait()

            @pl.when(fl >= 2)
            def _():
                lax.cond(lax.rem(fl, 2) == 0,
                         lambda: drain_prev(0), lambda: drain_prev(1))

            plsc.subcore_barrier()
            for k in range(CH // (128 * NS)):
                r = s * (CH // NS) + k * 128
                pltpu.sync_copy(agg_sp.at[pl.ds(r, 128)],
                                agg_hbm.at[pl.ds(base + r, 128)])
            plsc.subcore_barrier()

        return _

    lax.fori_loop(0, NCHUNK, chunk_body, None)


_deg_call = functools.partial(
    pl.kernel, _deg_body,
    out_type=jax.ShapeDtypeStruct((EGRP, 2, NPH), jnp.float32),
    mesh=_mesh,
    compiler_params=_sc_params,
    scratch_types=[
        pltpu.VMEM((BRD, 128), jnp.int32),
        pltpu.VMEM((BRD, 128), jnp.int32),
        pltpu.VMEM((DEG_SH,), jnp.float32),
        pltpu.VMEM((DEG_SH,), jnp.float32),
    ],
)()

_msg_call = functools.partial(
    pl.kernel, _msg_body,
    out_type=jax.ShapeDtypeStruct((AGGR, DF), jnp.float32),
    mesh=_mesh,
    compiler_params=_sc_params,
    scratch_types=[
        pltpu.VMEM((BR, 128), jnp.int32),
        pltpu.VMEM((BR, 128), jnp.int32),
        pltpu.VMEM((QCAP,), jnp.int32),
        pltpu.VMEM((QCAP,), jnp.int32),
        pltpu.VMEM((128,), jnp.int32),
        pltpu.VMEM((128,), jnp.int32),
        pltpu.VMEM((128,), jnp.int32),
        pltpu.VMEM((128,), jnp.int32),
        pltpu.VMEM((128, DF), jnp.float32),
        pltpu.VMEM((128, DF), jnp.float32),
        pltpu.VMEM((64, DF), jnp.float32),
        pltpu.VMEM_SHARED((CH2, DF), jnp.float32),
        pltpu.SemaphoreType.DMA,
        pltpu.SemaphoreType.DMA,
        pltpu.SemaphoreType.DMA,
        pltpu.SemaphoreType.DMA,
    ],
)()


def _prep_body(feat_ref, *refs):
    deg_refs = refs[:2 * EGRP]
    h_ref, nd_ref = refs[2 * EGRP:]
    x0 = feat_ref[0]
    x1 = feat_ref[1]
    do = deg_refs[0][...]
    di = deg_refs[EGRP][...]
    for e in range(1, EGRP):
        do = do + deg_refs[e][...]
        di = di + deg_refs[EGRP + e][...]
    ns = jnp.where(do > 0, lax.rsqrt(jnp.maximum(do, 1.0)), 0.0)
    nd = jnp.where(di > 0, lax.rsqrt(jnp.maximum(di, 1.0)), 0.0)
    h_ref[:, 0:D_IN] = x0 * ns[:, None]
    h_ref[:, D_IN:DF] = x1 * ns[:, None]
    nd_ref[...] = nd


def _read_body(agg_ref, nd_ref, wc, bc, w0, b0, w1, b1, w2, b2, out_ref):
    nd = nd_ref[...][:, None]

    def mlp(a):
        y = jnp.maximum(jnp.dot(a, wc[...], preferred_element_type=jnp.float32) + bc[...], 0.0)
        y = jnp.maximum(jnp.dot(y, w0[...], preferred_element_type=jnp.float32) + b0[...], 0.0)
        y = jnp.maximum(jnp.dot(y, w1[...], preferred_element_type=jnp.float32) + b1[...], 0.0)
        return jnp.dot(y, w2[...], preferred_element_type=jnp.float32) + b2[...]

    out_ref[0] = mlp(agg_ref[:, 0:D_IN] * nd)
    out_ref[1] = mlp(agg_ref[:, D_IN:DF] * nd)


def _full(shape):
    return pl.BlockSpec(shape, lambda i: tuple(0 for _ in shape))


_prep_call = pl.pallas_call(
    _prep_body,
    grid=(NBLK,),
    in_specs=[pl.BlockSpec((B, BN, D_IN), lambda i: (0, i, 0))]
    + [pl.BlockSpec((BN,), lambda i: (i,)) for _ in range(2 * EGRP)],
    out_specs=[
        pl.BlockSpec((BN, DF), lambda i: (i, 0)),
        pl.BlockSpec((BN,), lambda i: (i,)),
    ],
    out_shape=[
        jax.ShapeDtypeStruct((NPH, DF), jnp.float32),
        jax.ShapeDtypeStruct((NPH,), jnp.float32),
    ],
)

_read_call = pl.pallas_call(
    _read_body,
    grid=(NBLK,),
    in_specs=[
        pl.BlockSpec((BN, DF), lambda i: (i, 0)),
        pl.BlockSpec((BN,), lambda i: (i,)),
        _full((D_IN, D_H)),
        _full((D_H,)),
        _full((D_H, D_L0)),
        _full((D_L0,)),
        _full((D_L0, D_H)),
        _full((D_H,)),
        _full((D_H, D_OUT)),
        _full((D_OUT,)),
    ],
    out_specs=pl.BlockSpec((B, BN, D_OUT), lambda i: (0, i, 0)),
    out_shape=jax.ShapeDtypeStruct((B, NPH, D_OUT), jnp.float32),
)


def kernel(feat, edge_index, W_conv, b_conv, W0, b0, W1, b1, W2, b2):
    src = edge_index[0]
    dst = edge_index[1]
    pad = N + 1 + (jnp.arange(EPAD, dtype=jnp.int32) % 300)
    srcp = jnp.concatenate([src, pad]).reshape(ER, 128)
    dstp = jnp.concatenate([dst, pad]).reshape(ER, 128)
    xf = feat.reshape(B, N, D_IN)
    featp = jnp.pad(xf, ((0, 0), (0, NPH - N), (0, 0)))

    degs = _deg_call(srcp, dstp)
    deg_parts = ([degs[e, 0] for e in range(EGRP)]
                 + [degs[e, 1] for e in range(EGRP)])
    h, nd = _prep_call(featp, *deg_parts)
    agg = _msg_call(srcp, dstp, h)
    out = _read_call(agg, nd, W_conv, b_conv, W0, b0, W1, b1, W2, b2)
    return out[:, :N, :].reshape(B, N, 6, 4)


# speedup vs baseline: 92.4003x; 1.2929x over previous
import functools

import jax
import jax.numpy as jnp
from jax import lax
from jax.experimental import pallas as pl
from jax.experimental.pallas import tpu as pltpu
from jax.experimental.pallas import tpu_sc as plsc

B = 2
N = 100000
E = 1600000
D_IN = 48
D_H = 48
D_L0 = 96
D_OUT = 24
DF = 96

NC = 2
NS = 16
NPH = 100352
ER = 12544
EPAD = ER * 128 - E
CH = 12544
NCHUNK = 8
AGGR = NCHUNK * CH
CH2 = CH + 64
EGRP = 8
NRNG = 4
DEG_SH = NPH // NRNG
ROWS_EG = ER // EGRP
BRD = 112
EROWS_MSG = ER // NS
BR = 56
QCAP = 256
NBLK = 196
BN = 512

_mesh = plsc.VectorSubcoreMesh(core_axis_name="c", subcore_axis_name="s")
_sc_params = pltpu.CompilerParams(
    needs_layout_passes=False, use_tc_tiling_on_sc=False)


def _deg_body(srcp, dstp, out_hbm, sbuf, dbuf, tab_o, tab_i):
    c = lax.axis_index("c")
    s = lax.axis_index("s")
    t = c * NS + s
    eg = lax.rem(t, EGRP)
    nbase = (t // EGRP) * DEG_SH
    zv = jnp.zeros((16,), jnp.float32)
    ones_v = jnp.full((16,), 1.0, jnp.float32)

    def zloop(k, _):
        tab_o[pl.ds(k * 16, 16)] = zv
        tab_i[pl.ds(k * 16, 16)] = zv
        return _

    lax.fori_loop(0, DEG_SH // 16, zloop, None)

    def bloop(b, _):
        row0 = eg * ROWS_EG + b * BRD
        pltpu.sync_copy(srcp.at[pl.ds(row0, BRD)], sbuf)
        pltpu.sync_copy(dstp.at[pl.ds(row0, BRD)], dbuf)

        def row(j, _):
            def grp(g, _):
                sv = sbuf[j, pl.ds(g * 16, 16)]
                lo = sv - nbase
                mo = (lo >= 0) & (lo < DEG_SH)
                plsc.addupdate_scatter(tab_o, [lo], ones_v, mask=mo)
                dv = dbuf[j, pl.ds(g * 16, 16)]
                li = dv - nbase
                mi = (li >= 0) & (li < DEG_SH)
                plsc.addupdate_scatter(tab_i, [li], ones_v, mask=mi)
                return _

            return lax.fori_loop(0, 8, grp, None)

        return lax.fori_loop(0, BRD, row, None)

    lax.fori_loop(0, ROWS_EG // BRD, bloop, None)
    pltpu.sync_copy(tab_o, out_hbm.at[eg, 0, pl.ds(nbase, DEG_SH)])
    pltpu.sync_copy(tab_i, out_hbm.at[eg, 1, pl.ds(nbase, DEG_SH)])


def _msg_body(srcp, dstp, h_hbm, agg_hbm,
              sbuf, dbuf, qs, qd, sia, sib, dia, dib, ra, rb, zb, agg_sp,
              gsa, gsb, ssa, ssb):
    c = lax.axis_index("c")
    s = lax.axis_index("s")
    zv = jnp.zeros((16,), jnp.float32)

    def zb_init(i, _):
        zb[i // 6, pl.ds(lax.rem(i, 6) * 16, 16)] = zv
        return _

    lax.fori_loop(0, 64 * 6, zb_init, None)

    sibufs = (sia, sib)
    dibufs = (dia, dib)
    rows = (ra, rb)
    gsems = (gsa, gsb)
    ssems = (ssa, ssb)

    def flush_slot(p, fl, cnt):
        o = 1 - p

        @pl.when(fl >= 1)
        def _():
            pltpu.make_async_copy(h_hbm.at[sibufs[o]], rows[o],
                                  gsems[o]).wait()
            pltpu.async_copy(rows[o], agg_sp.at[dibufs[o]], ssems[o],
                             add=True)

        @pl.when(fl >= 2)
        def _():
            pltpu.make_async_copy(rows[p], agg_sp.at[dibufs[p]],
                                  ssems[p]).wait()

        for i in range(8):
            sibufs[p][pl.ds(i * 16, 16)] = qs[pl.ds(i * 16, 16)]
            dibufs[p][pl.ds(i * 16, 16)] = qd[pl.ds(i * 16, 16)]
        pltpu.async_copy(h_hbm.at[sibufs[p]], rows[p], gsems[p])
        for i in range(8):
            qs[pl.ds(i * 16, 16)] = qs[pl.ds(128 + i * 16, 16)]
            qd[pl.ds(i * 16, 16)] = qd[pl.ds(128 + i * 16, 16)]
        return cnt - 128, fl + 1

    def do_flush(state):
        cnt, fl = state
        return lax.cond(lax.rem(fl, 2) == 0,
                        lambda a, b: flush_slot(0, a, b),
                        lambda a, b: flush_slot(1, a, b),
                        fl, cnt)

    def chunk_body(c7, _):
        @pl.when(lax.rem(c7, 2) == c)
        def _run():
            base = c7 * CH
            z0 = s * (CH // NS)
            for k in range(12):
                pltpu.sync_copy(zb, agg_sp.at[pl.ds(z0 + k * 64, 64)])
            pltpu.sync_copy(zb.at[pl.ds(0, 16)],
                            agg_sp.at[pl.ds(z0 + 768, 16)])
            plsc.subcore_barrier()

            def stage(bk, state):
                row0 = s * EROWS_MSG + bk * BR
                pltpu.sync_copy(srcp.at[pl.ds(row0, BR)], sbuf)
                pltpu.sync_copy(dstp.at[pl.ds(row0, BR)], dbuf)

                def row(j, state):
                    def grp(g, cnt):
                        sv = sbuf[j, pl.ds(g * 16, 16)]
                        dv = dbuf[j, pl.ds(g * 16, 16)]
                        m = (dv >= base) & (dv < base + CH)
                        dl = dv - base
                        plsc.store_compressed(qs.at[pl.ds(cnt, 16)], sv,
                                              mask=m)
                        plsc.store_compressed(qd.at[pl.ds(cnt, 16)], dl,
                                              mask=m)
                        return cnt + plsc.all_reduce_population_count(m)[0]

                    cnt, fl = state
                    cnt = lax.fori_loop(0, 8, grp, cnt)
                    return lax.cond(cnt >= 128, do_flush,
                                    lambda x: x, (cnt, fl))

                return lax.fori_loop(0, BR, row, state)

            state = lax.fori_loop(0, EROWS_MSG // BR, stage,
                                  (jnp.int32(0), jnp.int32(0)))
            cnt, fl = state
            pad_s = jnp.zeros((16,), jnp.int32)
            pad_d = jnp.full((16,), CH, jnp.int32)
            for i in range(8):
                qs[pl.ds(cnt + i * 16, 16)] = pad_s
                qd[pl.ds(cnt + i * 16, 16)] = pad_d
            _, fl = do_flush((cnt, fl))

            def drain_last(p):
                pltpu.make_async_copy(h_hbm.at[sibufs[p]], rows[p],
                                      gsems[p]).wait()
                pltpu.async_copy(rows[p], agg_sp.at[dibufs[p]], ssems[p],
                                 add=True)
                pltpu.make_async_copy(rows[p], agg_sp.at[dibufs[p]],
                                      ssems[p]).wait()

            lax.cond(lax.rem(fl, 2) == 1,
                     lambda: drain_last(0), lambda: drain_last(1))

            def drain_prev(p):
                pltpu.make_async_copy(rows[p], agg_sp.at[dibufs[p]],
                                      ssems[p]).wait()

            @pl.when(fl >= 2)
            def _():
                lax.cond(lax.rem(fl, 2) == 0,
                         lambda: drain_prev(0), lambda: drain_prev(1))

            plsc.subcore_barrier()
            for k in range(6):
                r = z0 + k * 128
                pltpu.sync_copy(agg_sp.at[pl.ds(r, 128)],
                                agg_hbm.at[pl.ds(base + r, 128)])
            pltpu.sync_copy(agg_sp.at[pl.ds(z0 + 768, 16)],
                            agg_hbm.at[pl.ds(base + z0 + 768, 16)])
            plsc.subcore_barrier()

        return _

    lax.fori_loop(0, NCHUNK, chunk_body, None)


_deg_call = functools.partial(
    pl.kernel, _deg_body,
    out_type=jax.ShapeDtypeStruct((EGRP, 2, NPH), jnp.float32),
    mesh=_mesh,
    compiler_params=_sc_params,
    scratch_types=[
        pltpu.VMEM((BRD, 128), jnp.int32),
        pltpu.VMEM((BRD, 128), jnp.int32),
        pltpu.VMEM((DEG_SH,), jnp.float32),
        pltpu.VMEM((DEG_SH,), jnp.float32),
    ],
)()

_msg_call = functools.partial(
    pl.kernel, _msg_body,
    out_type=jax.ShapeDtypeStruct((AGGR, DF), jnp.float32),
    mesh=_mesh,
    compiler_params=_sc_params,
    scratch_types=[
        pltpu.VMEM((BR, 128), jnp.int32),
        pltpu.VMEM((BR, 128), jnp.int32),
        pltpu.VMEM((QCAP,), jnp.int32),
        pltpu.VMEM((QCAP,), jnp.int32),
        pltpu.VMEM((128,), jnp.int32),
        pltpu.VMEM((128,), jnp.int32),
        pltpu.VMEM((128,), jnp.int32),
        pltpu.VMEM((128,), jnp.int32),
        pltpu.VMEM((128, DF), jnp.float32),
        pltpu.VMEM((128, DF), jnp.float32),
        pltpu.VMEM((64, DF), jnp.float32),
        pltpu.VMEM_SHARED((CH2, DF), jnp.float32),
        pltpu.SemaphoreType.DMA,
        pltpu.SemaphoreType.DMA,
        pltpu.SemaphoreType.DMA,
        pltpu.SemaphoreType.DMA,
    ],
)()


def _prep_body(feat_ref, *refs):
    deg_refs = refs[:2 * EGRP]
    h_ref, nd_ref = refs[2 * EGRP:]
    x0 = feat_ref[0]
    x1 = feat_ref[1]
    do = deg_refs[0][...]
    di = deg_refs[EGRP][...]
    for e in range(1, EGRP):
        do = do + deg_refs[e][...]
        di = di + deg_refs[EGRP + e][...]
    ns = jnp.where(do > 0, lax.rsqrt(jnp.maximum(do, 1.0)), 0.0)
    nd = jnp.where(di > 0, lax.rsqrt(jnp.maximum(di, 1.0)), 0.0)
    h_ref[:, 0:D_IN] = x0 * ns[:, None]
    h_ref[:, D_IN:DF] = x1 * ns[:, None]
    nd_ref[...] = nd


def _read_body(agg_ref, nd_ref, wc, bc, w0, b0, w1, b1, w2, b2, out_ref):
    nd = nd_ref[...][:, None]

    def mlp(a):
        y = jnp.maximum(jnp.dot(a, wc[...], preferred_element_type=jnp.float32) + bc[...], 0.0)
        y = jnp.maximum(jnp.dot(y, w0[...], preferred_element_type=jnp.float32) + b0[...], 0.0)
        y = jnp.maximum(jnp.dot(y, w1[...], preferred_element_type=jnp.float32) + b1[...], 0.0)
        return jnp.dot(y, w2[...], preferred_element_type=jnp.float32) + b2[...]

    out_ref[0] = mlp(agg_ref[:, 0:D_IN] * nd)
    out_ref[1] = mlp(agg_ref[:, D_IN:DF] * nd)


def _full(shape):
    return pl.BlockSpec(shape, lambda i: tuple(0 for _ in shape))


_prep_call = pl.pallas_call(
    _prep_body,
    grid=(NBLK,),
    in_specs=[pl.BlockSpec((B, BN, D_IN), lambda i: (0, i, 0))]
    + [pl.BlockSpec((BN,), lambda i: (i,)) for _ in range(2 * EGRP)],
    out_specs=[
        pl.BlockSpec((BN, DF), lambda i: (i, 0)),
        pl.BlockSpec((BN,), lambda i: (i,)),
    ],
    out_shape=[
        jax.ShapeDtypeStruct((NPH, DF), jnp.float32),
        jax.ShapeDtypeStruct((NPH,), jnp.float32),
    ],
)

_read_call = pl.pallas_call(
    _read_body,
    grid=(NBLK,),
    in_specs=[
        pl.BlockSpec((BN, DF), lambda i: (i, 0)),
        pl.BlockSpec((BN,), lambda i: (i,)),
        _full((D_IN, D_H)),
        _full((D_H,)),
        _full((D_H, D_L0)),
        _full((D_L0,)),
        _full((D_L0, D_H)),
        _full((D_H,)),
        _full((D_H, D_OUT)),
        _full((D_OUT,)),
    ],
    out_specs=pl.BlockSpec((B, BN, D_OUT), lambda i: (0, i, 0)),
    out_shape=jax.ShapeDtypeStruct((B, NPH, D_OUT), jnp.float32),
)


def kernel(feat, edge_index, W_conv, b_conv, W0, b0, W1, b1, W2, b2):
    src = edge_index[0]
    dst = edge_index[1]
    pad = N + 1 + (jnp.arange(EPAD, dtype=jnp.int32) % 300)
    srcp = jnp.concatenate([src, pad]).reshape(ER, 128)
    dstp = jnp.concatenate([dst, pad]).reshape(ER, 128)
    xf = feat.reshape(B, N, D_IN)
    featp = jnp.pad(xf, ((0, 0), (0, NPH - N), (0, 0)))

    degs = _deg_call(srcp, dstp)
    deg_parts = ([degs[e, 0] for e in range(EGRP)]
                 + [degs[e, 1] for e in range(EGRP)])
    h, nd = _prep_call(featp, *deg_parts)
    agg = _msg_call(srcp, dstp, h)
    out = _read_call(agg, nd, W_conv, b_conv, W0, b0, W1, b1, W2, b2)
    return out[:, :N, :].reshape(B, N, 6, 4)


# speedup vs baseline: 97.5527x; 1.0558x over previous
import functools

import jax
import jax.numpy as jnp
from jax import lax
from jax.experimental import pallas as pl
from jax.experimental.pallas import tpu as pltpu
from jax.experimental.pallas import tpu_sc as plsc

B = 2
N = 100000
E = 1600000
D_IN = 48
D_H = 48
D_L0 = 96
D_OUT = 24
DF = 96

NC = 2
NS = 16
NPH = 100352
ER = 12544
EPAD = ER * 128 - E
CH = 12544
NCHUNK = 8
AGGR = NCHUNK * CH
CH2 = CH + 64
EGRP = 8
NRNG = 4
DEG_SH = NPH // NRNG
ROWS_EG = ER // EGRP
BRD = 112
EROWS_MSG = ER // NS
BR = 16
QCAP = 256
NBLK = 196
BN = 512

_mesh = plsc.VectorSubcoreMesh(core_axis_name="c", subcore_axis_name="s")
_sc_params = pltpu.CompilerParams(
    needs_layout_passes=False, use_tc_tiling_on_sc=False)


def _deg_body(srcp, dstp, out_hbm, sba, sbb, dba, dbb, tab_o, tab_i,
              sta, stb):
    c = lax.axis_index("c")
    s = lax.axis_index("s")
    t = c * NS + s
    eg = lax.rem(t, EGRP)
    nbase = (t // EGRP) * DEG_SH
    zv = jnp.zeros((16,), jnp.float32)
    ones_v = jnp.full((16,), 1.0, jnp.float32)
    sbufs = (sba, sbb)
    dbufs = (dba, dbb)
    sems = (sta, stb)
    nblocks = ROWS_EG // BRD

    def zloop(k, _):
        tab_o[pl.ds(k * 16, 16)] = zv
        tab_i[pl.ds(k * 16, 16)] = zv
        return _

    lax.fori_loop(0, DEG_SH // 16, zloop, None)

    def fire(b, p):
        row0 = eg * ROWS_EG + b * BRD
        pltpu.async_copy(srcp.at[pl.ds(row0, BRD)], sbufs[p], sems[p])
        pltpu.async_copy(dstp.at[pl.ds(row0, BRD)], dbufs[p], sems[p])

    def block_slot(p, b):
        row0 = eg * ROWS_EG + b * BRD
        pltpu.make_async_copy(srcp.at[pl.ds(row0, BRD)], sbufs[p],
                              sems[p]).wait()
        pltpu.make_async_copy(dstp.at[pl.ds(row0, BRD)], dbufs[p],
                              sems[p]).wait()

        @pl.when(b + 1 < nblocks)
        def _():
            fire(b + 1, 1 - p)

        sbuf = sbufs[p]
        dbuf = dbufs[p]

        def row(j, _):
            def grp(g, _):
                sv = sbuf[j, pl.ds(g * 16, 16)]
                lo = sv - nbase
                mo = (lo >= 0) & (lo < DEG_SH)
                plsc.addupdate_scatter(tab_o, [lo], ones_v, mask=mo)
                dv = dbuf[j, pl.ds(g * 16, 16)]
                li = dv - nbase
                mi = (li >= 0) & (li < DEG_SH)
                plsc.addupdate_scatter(tab_i, [li], ones_v, mask=mi)
                return _

            return lax.fori_loop(0, 8, grp, None)

        return lax.fori_loop(0, BRD, row, None)

    fire(0, 0)

    def bloop(b, _):
        lax.cond(lax.rem(b, 2) == 0,
                 lambda x: block_slot(0, x),
                 lambda x: block_slot(1, x), b)
        return _

    lax.fori_loop(0, nblocks, bloop, None)
    pltpu.sync_copy(tab_o, out_hbm.at[eg, 0, pl.ds(nbase, DEG_SH)])
    pltpu.sync_copy(tab_i, out_hbm.at[eg, 1, pl.ds(nbase, DEG_SH)])


def _msg_body(srcp, dstp, h_hbm, agg_hbm,
              sb0, sb1, db0, db1, qs, qd, sia, sib, dia, dib, ra, rb, zb,
              agg_sp, gsa, gsb, ssa, ssb, st0, st1):
    c = lax.axis_index("c")
    s = lax.axis_index("s")
    zv = jnp.zeros((16,), jnp.float32)

    def zb_init(i, _):
        zb[i // 6, pl.ds(lax.rem(i, 6) * 16, 16)] = zv
        return _

    lax.fori_loop(0, 64 * 6, zb_init, None)

    sibufs = (sia, sib)
    dibufs = (dia, dib)
    rows = (ra, rb)
    gsems = (gsa, gsb)
    ssems = (ssa, ssb)
    sbufs = (sb0, sb1)
    dbufs = (db0, db1)
    stsems = (st0, st1)

    def flush_slot(p, fl, cnt):
        o = 1 - p

        @pl.when(fl >= 1)
        def _():
            pltpu.make_async_copy(h_hbm.at[sibufs[o]], rows[o],
                                  gsems[o]).wait()
            pltpu.async_copy(rows[o], agg_sp.at[dibufs[o]], ssems[o],
                             add=True)

        @pl.when(fl >= 2)
        def _():
            pltpu.make_async_copy(rows[p], agg_sp.at[dibufs[p]],
                                  ssems[p]).wait()

        for i in range(8):
            sibufs[p][pl.ds(i * 16, 16)] = qs[pl.ds(i * 16, 16)]
            dibufs[p][pl.ds(i * 16, 16)] = qd[pl.ds(i * 16, 16)]
        pltpu.async_copy(h_hbm.at[sibufs[p]], rows[p], gsems[p])
        for i in range(8):
            qs[pl.ds(i * 16, 16)] = qs[pl.ds(128 + i * 16, 16)]
            qd[pl.ds(i * 16, 16)] = qd[pl.ds(128 + i * 16, 16)]
        return cnt - 128, fl + 1

    def do_flush(state):
        cnt, fl = state
        return lax.cond(lax.rem(fl, 2) == 0,
                        lambda a, b: flush_slot(0, a, b),
                        lambda a, b: flush_slot(1, a, b),
                        fl, cnt)

    def chunk_body(c7, _):
        @pl.when(lax.rem(c7, 2) == c)
        def _run():
            base = c7 * CH
            z0 = s * (CH // NS)
            for k in range(12):
                pltpu.sync_copy(zb, agg_sp.at[pl.ds(z0 + k * 64, 64)])
            pltpu.sync_copy(zb.at[pl.ds(0, 16)],
                            agg_sp.at[pl.ds(z0 + 768, 16)])
            plsc.subcore_barrier()

            nstages = EROWS_MSG // BR

            def fire_stage(bk, p):
                row0 = s * EROWS_MSG + bk * BR
                pltpu.async_copy(srcp.at[pl.ds(row0, BR)], sbufs[p],
                                 stsems[p])
                pltpu.async_copy(dstp.at[pl.ds(row0, BR)], dbufs[p],
                                 stsems[p])

            def stage_slot(sp, bk, state):
                row0 = s * EROWS_MSG + bk * BR
                pltpu.make_async_copy(srcp.at[pl.ds(row0, BR)], sbufs[sp],
                                      stsems[sp]).wait()
                pltpu.make_async_copy(dstp.at[pl.ds(row0, BR)], dbufs[sp],
                                      stsems[sp]).wait()

                @pl.when(bk + 1 < nstages)
                def _():
                    fire_stage(bk + 1, 1 - sp)

                sbuf = sbufs[sp]
                dbuf = dbufs[sp]

                def row(j, state):
                    def grp(g, cnt):
                        sv = sbuf[j, pl.ds(g * 16, 16)]
                        dv = dbuf[j, pl.ds(g * 16, 16)]
                        m = (dv >= base) & (dv < base + CH)
                        dl = dv - base
                        plsc.store_compressed(qs.at[pl.ds(cnt, 16)], sv,
                                              mask=m)
                        plsc.store_compressed(qd.at[pl.ds(cnt, 16)], dl,
                                              mask=m)
                        return cnt + plsc.all_reduce_population_count(m)[0]

                    cnt, fl = state
                    cnt = lax.fori_loop(0, 8, grp, cnt)
                    return lax.cond(cnt >= 128, do_flush,
                                    lambda x: x, (cnt, fl))

                return lax.fori_loop(0, BR, row, state)

            def stage(bk, state):
                return lax.cond(lax.rem(bk, 2) == 0,
                                lambda b, st: stage_slot(0, b, st),
                                lambda b, st: stage_slot(1, b, st),
                                bk, state)

            fire_stage(0, 0)
            state = lax.fori_loop(0, EROWS_MSG // BR, stage,
                                  (jnp.int32(0), jnp.int32(0)))
            cnt, fl = state
            pad_s = jnp.zeros((16,), jnp.int32)
            pad_d = jnp.full((16,), CH, jnp.int32)
            for i in range(8):
                qs[pl.ds(cnt + i * 16, 16)] = pad_s
                qd[pl.ds(cnt + i * 16, 16)] = pad_d
            _, fl = do_flush((cnt, fl))

            def drain_last(p):
                pltpu.make_async_copy(h_hbm.at[sibufs[p]], rows[p],
                                      gsems[p]).wait()
                pltpu.async_copy(rows[p], agg_sp.at[dibufs[p]], ssems[p],
                                 add=True)
                pltpu.make_async_copy(rows[p], agg_sp.at[dibufs[p]],
                                      ssems[p]).wait()

            lax.cond(lax.rem(fl, 2) == 1,
                     lambda: drain_last(0), lambda: drain_last(1))

            def drain_prev(p):
                pltpu.make_async_copy(rows[p], agg_sp.at[dibufs[p]],
                                      ssems[p]).wait()

            @pl.when(fl >= 2)
            def _():
                lax.cond(lax.rem(fl, 2) == 0,
                         lambda: drain_prev(0), lambda: drain_prev(1))

            plsc.subcore_barrier()
            for k in range(6):
                r = z0 + k * 128
                pltpu.sync_copy(agg_sp.at[pl.ds(r, 128)],
                                agg_hbm.at[pl.ds(base + r, 128)])
            pltpu.sync_copy(agg_sp.at[pl.ds(z0 + 768, 16)],
                            agg_hbm.at[pl.ds(base + z0 + 768, 16)])
            plsc.subcore_barrier()

        return _

    lax.fori_loop(0, NCHUNK, chunk_body, None)


_deg_call = functools.partial(
    pl.kernel, _deg_body,
    out_type=jax.ShapeDtypeStruct((EGRP, 2, NPH), jnp.float32),
    mesh=_mesh,
    compiler_params=_sc_params,
    scratch_types=[
        pltpu.VMEM((BRD, 128), jnp.int32),
        pltpu.VMEM((BRD, 128), jnp.int32),
        pltpu.VMEM((BRD, 128), jnp.int32),
        pltpu.VMEM((BRD, 128), jnp.int32),
        pltpu.VMEM((DEG_SH,), jnp.float32),
        pltpu.VMEM((DEG_SH,), jnp.float32),
        pltpu.SemaphoreType.DMA,
        pltpu.SemaphoreType.DMA,
    ],
)()

_msg_call = functools.partial(
    pl.kernel, _msg_body,
    out_type=jax.ShapeDtypeStruct((AGGR, DF), jnp.float32),
    mesh=_mesh,
    compiler_params=_sc_params,
    scratch_types=[
        pltpu.VMEM((BR, 128), jnp.int32),
        pltpu.VMEM((BR, 128), jnp.int32),
        pltpu.VMEM((BR, 128), jnp.int32),
        pltpu.VMEM((BR, 128), jnp.int32),
        pltpu.VMEM((QCAP,), jnp.int32),
        pltpu.VMEM((QCAP,), jnp.int32),
        pltpu.VMEM((128,), jnp.int32),
        pltpu.VMEM((128,), jnp.int32),
        pltpu.VMEM((128,), jnp.int32),
        pltpu.VMEM((128,), jnp.int32),
        pltpu.VMEM((128, DF), jnp.float32),
        pltpu.VMEM((128, DF), jnp.float32),
        pltpu.VMEM((64, DF), jnp.float32),
        pltpu.VMEM_SHARED((CH2, DF), jnp.float32),
        pltpu.SemaphoreType.DMA,
        pltpu.SemaphoreType.DMA,
        pltpu.SemaphoreType.DMA,
        pltpu.SemaphoreType.DMA,
        pltpu.SemaphoreType.DMA,
        pltpu.SemaphoreType.DMA,
    ],
)()


def _prep_body(feat_ref, *refs):
    deg_refs = refs[:2 * EGRP]
    h_ref, nd_ref = refs[2 * EGRP:]
    x0 = feat_ref[0]
    x1 = feat_ref[1]
    do = deg_refs[0][...]
    di = deg_refs[EGRP][...]
    for e in range(1, EGRP):
        do = do + deg_refs[e][...]
        di = di + deg_refs[EGRP + e][...]
    ns = jnp.where(do > 0, lax.rsqrt(jnp.maximum(do, 1.0)), 0.0)
    nd = jnp.where(di > 0, lax.rsqrt(jnp.maximum(di, 1.0)), 0.0)
    h_ref[:, 0:D_IN] = x0 * ns[:, None]
    h_ref[:, D_IN:DF] = x1 * ns[:, None]
    nd_ref[...] = nd


def _read_body(agg_ref, nd_ref, wc, bc, w0, b0, w1, b1, w2, b2, out_ref):
    nd = nd_ref[...][:, None]

    def mlp(a):
        y = jnp.maximum(jnp.dot(a, wc[...], preferred_element_type=jnp.float32) + bc[...], 0.0)
        y = jnp.maximum(jnp.dot(y, w0[...], preferred_element_type=jnp.float32) + b0[...], 0.0)
        y = jnp.maximum(jnp.dot(y, w1[...], preferred_element_type=jnp.float32) + b1[...], 0.0)
        return jnp.dot(y, w2[...], preferred_element_type=jnp.float32) + b2[...]

    out_ref[0] = mlp(agg_ref[:, 0:D_IN] * nd)
    out_ref[1] = mlp(agg_ref[:, D_IN:DF] * nd)


def _full(shape):
    return pl.BlockSpec(shape, lambda i: tuple(0 for _ in shape))


_prep_call = pl.pallas_call(
    _prep_body,
    grid=(NBLK,),
    in_specs=[pl.BlockSpec((B, BN, D_IN), lambda i: (0, i, 0))]
    + [pl.BlockSpec((BN,), lambda i: (i,)) for _ in range(2 * EGRP)],
    out_specs=[
        pl.BlockSpec((BN, DF), lambda i: (i, 0)),
        pl.BlockSpec((BN,), lambda i: (i,)),
    ],
    out_shape=[
        jax.ShapeDtypeStruct((NPH, DF), jnp.float32),
        jax.ShapeDtypeStruct((NPH,), jnp.float32),
    ],
)

_read_call = pl.pallas_call(
    _read_body,
    grid=(NBLK,),
    in_specs=[
        pl.BlockSpec((BN, DF), lambda i: (i, 0)),
        pl.BlockSpec((BN,), lambda i: (i,)),
        _full((D_IN, D_H)),
        _full((D_H,)),
        _full((D_H, D_L0)),
        _full((D_L0,)),
        _full((D_L0, D_H)),
        _full((D_H,)),
        _full((D_H, D_OUT)),
        _full((D_OUT,)),
    ],
    out_specs=pl.BlockSpec((B, BN, D_OUT), lambda i: (0, i, 0)),
    out_shape=jax.ShapeDtypeStruct((B, NPH, D_OUT), jnp.float32),
)


def kernel(feat, edge_index, W_conv, b_conv, W0, b0, W1, b1, W2, b2):
    src = edge_index[0]
    dst = edge_index[1]
    pad = N + 1 + (jnp.arange(EPAD, dtype=jnp.int32) % 300)
    srcp = jnp.concatenate([src, pad]).reshape(ER, 128)
    dstp = jnp.concatenate([dst, pad]).reshape(ER, 128)
    xf = feat.reshape(B, N, D_IN)
    featp = jnp.pad(xf, ((0, 0), (0, NPH - N), (0, 0)))

    degs = _deg_call(srcp, dstp)
    deg_parts = ([degs[e, 0] for e in range(EGRP)]
                 + [degs[e, 1] for e in range(EGRP)])
    h, nd = _prep_call(featp, *deg_parts)
    agg = _msg_call(srcp, dstp, h)
    out = _read_call(agg, nd, W_conv, b_conv, W0, b0, W1, b1, W2, b2)
    return out[:, :N, :].reshape(B, N, 6, 4)


# speedup vs baseline: 107.7586x; 1.1046x over previous
import functools

import jax
import jax.numpy as jnp
from jax import lax
from jax.experimental import pallas as pl
from jax.experimental.pallas import tpu as pltpu
from jax.experimental.pallas import tpu_sc as plsc

B = 2
N = 100000
E = 1600000
D_IN = 48
D_H = 48
D_L0 = 96
D_OUT = 24
DF = 96

NC = 2
NS = 16
NPH = 100352
ER = 12544
EPAD = ER * 128 - E
CH = 12544
NCHUNK = 8
AGGR = NCHUNK * CH
CH2 = CH + 64
EGRP = 8
NRNG = 4
DEG_SH = NPH // NRNG
ROWS_EG = ER // EGRP
BRD = 112
EROWS_MSG = ER // NS
BR = 16
QCAP = 256
NBLK = 98
BN = 1024

_mesh = plsc.VectorSubcoreMesh(core_axis_name="c", subcore_axis_name="s")
_sc_params = pltpu.CompilerParams(
    needs_layout_passes=False, use_tc_tiling_on_sc=False)


def _deg_body(srcp, dstp, out_hbm, sba, sbb, dba, dbb, tab_o, tab_i,
              sta, stb):
    c = lax.axis_index("c")
    s = lax.axis_index("s")
    t = c * NS + s
    eg = lax.rem(t, EGRP)
    nbase = (t // EGRP) * DEG_SH
    zv = jnp.zeros((16,), jnp.float32)
    ones_v = jnp.full((16,), 1.0, jnp.float32)
    sbufs = (sba, sbb)
    dbufs = (dba, dbb)
    sems = (sta, stb)
    nblocks = ROWS_EG // BRD

    def zloop(k, _):
        tab_o[pl.ds(k * 16, 16)] = zv
        tab_i[pl.ds(k * 16, 16)] = zv
        return _

    lax.fori_loop(0, DEG_SH // 16, zloop, None)

    def fire(b, p):
        row0 = eg * ROWS_EG + b * BRD
        pltpu.async_copy(srcp.at[pl.ds(row0, BRD)], sbufs[p], sems[p])
        pltpu.async_copy(dstp.at[pl.ds(row0, BRD)], dbufs[p], sems[p])

    def block_slot(p, b):
        row0 = eg * ROWS_EG + b * BRD
        pltpu.make_async_copy(srcp.at[pl.ds(row0, BRD)], sbufs[p],
                              sems[p]).wait()
        pltpu.make_async_copy(dstp.at[pl.ds(row0, BRD)], dbufs[p],
                              sems[p]).wait()

        @pl.when(b + 1 < nblocks)
        def _():
            fire(b + 1, 1 - p)

        sbuf = sbufs[p]
        dbuf = dbufs[p]

        def row(j, _):
            def grp(g, _):
                sv = sbuf[j, pl.ds(g * 16, 16)]
                lo = sv - nbase
                mo = (lo >= 0) & (lo < DEG_SH)
                plsc.addupdate_scatter(tab_o, [lo], ones_v, mask=mo)
                dv = dbuf[j, pl.ds(g * 16, 16)]
                li = dv - nbase
                mi = (li >= 0) & (li < DEG_SH)
                plsc.addupdate_scatter(tab_i, [li], ones_v, mask=mi)
                return _

            return lax.fori_loop(0, 8, grp, None)

        return lax.fori_loop(0, BRD, row, None)

    fire(0, 0)

    def bloop(b, _):
        lax.cond(lax.rem(b, 2) == 0,
                 lambda x: block_slot(0, x),
                 lambda x: block_slot(1, x), b)
        return _

    lax.fori_loop(0, nblocks, bloop, None)
    pltpu.sync_copy(tab_o, out_hbm.at[eg, 0, pl.ds(nbase, DEG_SH)])
    pltpu.sync_copy(tab_i, out_hbm.at[eg, 1, pl.ds(nbase, DEG_SH)])


def _msg_body(srcp, dstp, h_hbm, agg_hbm,
              sb0, sb1, db0, db1, qs, qd, sia, sib, dia, dib, ra, rb, zb,
              agg_sp, gsa, gsb, ssa, ssb, st0, st1):
    c = lax.axis_index("c")
    s = lax.axis_index("s")
    zv = jnp.zeros((16,), jnp.float32)

    def zb_init(i, _):
        zb[i // 6, pl.ds(lax.rem(i, 6) * 16, 16)] = zv
        return _

    lax.fori_loop(0, 64 * 6, zb_init, None)

    sibufs = (sia, sib)
    dibufs = (dia, dib)
    rows = (ra, rb)
    gsems = (gsa, gsb)
    ssems = (ssa, ssb)
    sbufs = (sb0, sb1)
    dbufs = (db0, db1)
    stsems = (st0, st1)

    def flush_slot(p, fl, cnt):
        o = 1 - p

        @pl.when(fl >= 1)
        def _():
            pltpu.make_async_copy(h_hbm.at[sibufs[o]], rows[o],
                                  gsems[o]).wait()
            pltpu.async_copy(rows[o], agg_sp.at[dibufs[o]], ssems[o],
                             add=True)

        @pl.when(fl >= 2)
        def _():
            pltpu.make_async_copy(rows[p], agg_sp.at[dibufs[p]],
                                  ssems[p]).wait()

        for i in range(8):
            sibufs[p][pl.ds(i * 16, 16)] = qs[pl.ds(i * 16, 16)]
            dibufs[p][pl.ds(i * 16, 16)] = qd[pl.ds(i * 16, 16)]
        pltpu.async_copy(h_hbm.at[sibufs[p]], rows[p], gsems[p])
        for i in range(8):
            qs[pl.ds(i * 16, 16)] = qs[pl.ds(128 + i * 16, 16)]
            qd[pl.ds(i * 16, 16)] = qd[pl.ds(128 + i * 16, 16)]
        return cnt - 128, fl + 1

    def do_flush(state):
        cnt, fl = state
        return lax.cond(lax.rem(fl, 2) == 0,
                        lambda a, b: flush_slot(0, a, b),
                        lambda a, b: flush_slot(1, a, b),
                        fl, cnt)

    def chunk_body(c7, _):
        @pl.when(lax.rem(c7, 2) == c)
        def _run():
            base = c7 * CH
            z0 = s * (CH // NS)
            for k in range(12):
                pltpu.sync_copy(zb, agg_sp.at[pl.ds(z0 + k * 64, 64)])
            pltpu.sync_copy(zb.at[pl.ds(0, 16)],
                            agg_sp.at[pl.ds(z0 + 768, 16)])
            plsc.subcore_barrier()

            nstages = EROWS_MSG // BR

            def fire_stage(bk, p):
                row0 = s * EROWS_MSG + bk * BR
                pltpu.async_copy(srcp.at[pl.ds(row0, BR)], sbufs[p],
                                 stsems[p])
                pltpu.async_copy(dstp.at[pl.ds(row0, BR)], dbufs[p],
                                 stsems[p])

            def stage_slot(sp, bk, state):
                row0 = s * EROWS_MSG + bk * BR
                pltpu.make_async_copy(srcp.at[pl.ds(row0, BR)], sbufs[sp],
                                      stsems[sp]).wait()
                pltpu.make_async_copy(dstp.at[pl.ds(row0, BR)], dbufs[sp],
                                      stsems[sp]).wait()

                @pl.when(bk + 1 < nstages)
                def _():
                    fire_stage(bk + 1, 1 - sp)

                sbuf = sbufs[sp]
                dbuf = dbufs[sp]

                def row(j, state):
                    def grp(g, cnt):
                        sv = sbuf[j, pl.ds(g * 16, 16)]
                        dv = dbuf[j, pl.ds(g * 16, 16)]
                        m = (dv >= base) & (dv < base + CH)
                        dl = dv - base
                        plsc.store_compressed(qs.at[pl.ds(cnt, 16)], sv,
                                              mask=m)
                        plsc.store_compressed(qd.at[pl.ds(cnt, 16)], dl,
                                              mask=m)
                        return cnt + plsc.all_reduce_population_count(m)[0]

                    cnt, fl = state
                    cnt = lax.fori_loop(0, 8, grp, cnt)
                    return lax.cond(cnt >= 128, do_flush,
                                    lambda x: x, (cnt, fl))

                return lax.fori_loop(0, BR, row, state)

            def stage(bk, state):
                return lax.cond(lax.rem(bk, 2) == 0,
                                lambda b, st: stage_slot(0, b, st),
                                lambda b, st: stage_slot(1, b, st),
                                bk, state)

            fire_stage(0, 0)
            state = lax.fori_loop(0, EROWS_MSG // BR, stage,
                                  (jnp.int32(0), jnp.int32(0)))
            cnt, fl = state
            pad_s = jnp.zeros((16,), jnp.int32)
            pad_d = jnp.full((16,), CH, jnp.int32)
            for i in range(8):
                qs[pl.ds(cnt + i * 16, 16)] = pad_s
                qd[pl.ds(cnt + i * 16, 16)] = pad_d
            _, fl = do_flush((cnt, fl))

            def drain_last(p):
                pltpu.make_async_copy(h_hbm.at[sibufs[p]], rows[p],
                                      gsems[p]).wait()
                pltpu.async_copy(rows[p], agg_sp.at[dibufs[p]], ssems[p],
                                 add=True)
                pltpu.make_async_copy(rows[p], agg_sp.at[dibufs[p]],
                                      ssems[p]).wait()

            lax.cond(lax.rem(fl, 2) == 1,
                     lambda: drain_last(0), lambda: drain_last(1))

            def drain_prev(p):
                pltpu.make_async_copy(rows[p], agg_sp.at[dibufs[p]],
                                      ssems[p]).wait()

            @pl.when(fl >= 2)
            def _():
                lax.cond(lax.rem(fl, 2) == 0,
                         lambda: drain_prev(0), lambda: drain_prev(1))

            plsc.subcore_barrier()
            for k in range(6):
                r = z0 + k * 128
                pltpu.sync_copy(agg_sp.at[pl.ds(r, 128)],
                                agg_hbm.at[pl.ds(base + r, 128)])
            pltpu.sync_copy(agg_sp.at[pl.ds(z0 + 768, 16)],
                            agg_hbm.at[pl.ds(base + z0 + 768, 16)])
            plsc.subcore_barrier()

        return _

    lax.fori_loop(0, NCHUNK, chunk_body, None)


_deg_call = functools.partial(
    pl.kernel, _deg_body,
    out_type=jax.ShapeDtypeStruct((EGRP, 2, NPH), jnp.float32),
    mesh=_mesh,
    compiler_params=_sc_params,
    scratch_types=[
        pltpu.VMEM((BRD, 128), jnp.int32),
        pltpu.VMEM((BRD, 128), jnp.int32),
        pltpu.VMEM((BRD, 128), jnp.int32),
        pltpu.VMEM((BRD, 128), jnp.int32),
        pltpu.VMEM((DEG_SH,), jnp.float32),
        pltpu.VMEM((DEG_SH,), jnp.float32),
        pltpu.SemaphoreType.DMA,
        pltpu.SemaphoreType.DMA,
    ],
)()

_msg_call = functools.partial(
    pl.kernel, _msg_body,
    out_type=jax.ShapeDtypeStruct((AGGR, DF), jnp.float32),
    mesh=_mesh,
    compiler_params=_sc_params,
    scratch_types=[
        pltpu.VMEM((BR, 128), jnp.int32),
        pltpu.VMEM((BR, 128), jnp.int32),
        pltpu.VMEM((BR, 128), jnp.int32),
        pltpu.VMEM((BR, 128), jnp.int32),
        pltpu.VMEM((QCAP,), jnp.int32),
        pltpu.VMEM((QCAP,), jnp.int32),
        pltpu.VMEM((128,), jnp.int32),
        pltpu.VMEM((128,), jnp.int32),
        pltpu.VMEM((128,), jnp.int32),
        pltpu.VMEM((128,), jnp.int32),
        pltpu.VMEM((128, DF), jnp.float32),
        pltpu.VMEM((128, DF), jnp.float32),
        pltpu.VMEM((64, DF), jnp.float32),
        pltpu.VMEM_SHARED((CH2, DF), jnp.float32),
        pltpu.SemaphoreType.DMA,
        pltpu.SemaphoreType.DMA,
        pltpu.SemaphoreType.DMA,
        pltpu.SemaphoreType.DMA,
        pltpu.SemaphoreType.DMA,
        pltpu.SemaphoreType.DMA,
    ],
)()


def _prep_body(feat_ref, *refs):
    deg_refs = refs[:2 * EGRP]
    h_ref, nd_ref = refs[2 * EGRP:]
    x0 = feat_ref[0]
    x1 = feat_ref[1]
    do = deg_refs[0][...]
    di = deg_refs[EGRP][...]
    for e in range(1, EGRP):
        do = do + deg_refs[e][...]
        di = di + deg_refs[EGRP + e][...]
    ns = jnp.where(do > 0, lax.rsqrt(jnp.maximum(do, 1.0)), 0.0)
    nd = jnp.where(di > 0, lax.rsqrt(jnp.maximum(di, 1.0)), 0.0)
    h_ref[:, 0:D_IN] = x0 * ns[:, None]
    h_ref[:, D_IN:DF] = x1 * ns[:, None]
    nd_ref[...] = nd


def _read_body(agg_ref, nd_ref, wc, bc, w0, b0, w1, b1, w2, b2, out_ref):
    nd = nd_ref[...][:, None]

    def mlp(a):
        y = jnp.maximum(jnp.dot(a, wc[...], preferred_element_type=jnp.float32) + bc[...], 0.0)
        y = jnp.maximum(jnp.dot(y, w0[...], preferred_element_type=jnp.float32) + b0[...], 0.0)
        y = jnp.maximum(jnp.dot(y, w1[...], preferred_element_type=jnp.float32) + b1[...], 0.0)
        return jnp.dot(y, w2[...], preferred_element_type=jnp.float32) + b2[...]

    out_ref[0] = mlp(agg_ref[:, 0:D_IN] * nd)
    out_ref[1] = mlp(agg_ref[:, D_IN:DF] * nd)


def _full(shape):
    return pl.BlockSpec(shape, lambda i: tuple(0 for _ in shape))


_prep_call = pl.pallas_call(
    _prep_body,
    grid=(NBLK,),
    in_specs=[pl.BlockSpec((B, BN, D_IN), lambda i: (0, i, 0))]
    + [pl.BlockSpec((BN,), lambda i: (i,)) for _ in range(2 * EGRP)],
    out_specs=[
        pl.BlockSpec((BN, DF), lambda i: (i, 0)),
        pl.BlockSpec((BN,), lambda i: (i,)),
    ],
    out_shape=[
        jax.ShapeDtypeStruct((NPH, DF), jnp.float32),
        jax.ShapeDtypeStruct((NPH,), jnp.float32),
    ],
)

_read_call = pl.pallas_call(
    _read_body,
    grid=(NBLK,),
    in_specs=[
        pl.BlockSpec((BN, DF), lambda i: (i, 0)),
        pl.BlockSpec((BN,), lambda i: (i,)),
        _full((D_IN, D_H)),
        _full((D_H,)),
        _full((D_H, D_L0)),
        _full((D_L0,)),
        _full((D_L0, D_H)),
        _full((D_H,)),
        _full((D_H, D_OUT)),
        _full((D_OUT,)),
    ],
    out_specs=pl.BlockSpec((B, BN, D_OUT), lambda i: (0, i, 0)),
    out_shape=jax.ShapeDtypeStruct((B, N, D_OUT), jnp.float32),
)


def kernel(feat, edge_index, W_conv, b_conv, W0, b0, W1, b1, W2, b2):
    src = edge_index[0]
    dst = edge_index[1]
    pad = N + 1 + (jnp.arange(EPAD, dtype=jnp.int32) % 300)
    srcp = jnp.concatenate([src, pad]).reshape(ER, 128)
    dstp = jnp.concatenate([dst, pad]).reshape(ER, 128)
    xf = feat.reshape(B, N, D_IN)
    featp = jnp.pad(xf, ((0, 0), (0, NPH - N), (0, 0)))

    degs = _deg_call(srcp, dstp)
    deg_parts = ([degs[e, 0] for e in range(EGRP)]
                 + [degs[e, 1] for e in range(EGRP)])
    h, nd = _prep_call(featp, *deg_parts)
    agg = _msg_call(srcp, dstp, h)
    out = _read_call(agg, nd, W_conv, b_conv, W0, b0, W1, b1, W2, b2)
    return out.reshape(B, N, 6, 4)


# speedup vs baseline: 107.7871x; 1.0003x over previous
import functools

import jax
import jax.numpy as jnp
from jax import lax
from jax.experimental import pallas as pl
from jax.experimental.pallas import tpu as pltpu
from jax.experimental.pallas import tpu_sc as plsc

B = 2
N = 100000
E = 1600000
D_IN = 48
D_H = 48
D_L0 = 96
D_OUT = 24
DF = 96

NC = 2
NS = 16
NPH = 100352
ER = 12544
EPAD = ER * 128 - E
CH = 12544
NCHUNK = 8
AGGR = NCHUNK * CH
CH2 = CH + 64
EGRP = 8
NRNG = 4
DEG_SH = NPH // NRNG
ROWS_EG = ER // EGRP
BRD = 112
EROWS_MSG = ER // NS
BR = 16
QCAP = 256
NBLK = 98
BN = 1024

_mesh = plsc.VectorSubcoreMesh(core_axis_name="c", subcore_axis_name="s")
_sc_params = pltpu.CompilerParams(
    needs_layout_passes=False, use_tc_tiling_on_sc=False)


def _deg_body(srcp, dstp, out_hbm, sba, sbb, dba, dbb, tab_o, tab_i,
              sta, stb):
    c = lax.axis_index("c")
    s = lax.axis_index("s")
    t = c * NS + s
    eg = lax.rem(t, EGRP)
    nbase = (t // EGRP) * DEG_SH
    zv = jnp.zeros((16,), jnp.float32)
    ones_v = jnp.full((16,), 1.0, jnp.float32)
    sbufs = (sba, sbb)
    dbufs = (dba, dbb)
    sems = (sta, stb)
    nblocks = ROWS_EG // BRD

    def zloop(k, _):
        tab_o[pl.ds(k * 16, 16)] = zv
        tab_i[pl.ds(k * 16, 16)] = zv
        return _

    lax.fori_loop(0, DEG_SH // 16, zloop, None)

    def fire(b, p):
        row0 = eg * ROWS_EG + b * BRD
        pltpu.async_copy(srcp.at[pl.ds(row0, BRD)], sbufs[p], sems[p])
        pltpu.async_copy(dstp.at[pl.ds(row0, BRD)], dbufs[p], sems[p])

    def block_slot(p, b):
        row0 = eg * ROWS_EG + b * BRD
        pltpu.make_async_copy(srcp.at[pl.ds(row0, BRD)], sbufs[p],
                              sems[p]).wait()
        pltpu.make_async_copy(dstp.at[pl.ds(row0, BRD)], dbufs[p],
                              sems[p]).wait()

        @pl.when(b + 1 < nblocks)
        def _():
            fire(b + 1, 1 - p)

        sbuf = sbufs[p]
        dbuf = dbufs[p]

        def row(j, _):
            def grp(g, _):
                sv = sbuf[j, pl.ds(g * 16, 16)]
                lo = sv - nbase
                mo = (lo >= 0) & (lo < DEG_SH)
                plsc.addupdate_scatter(tab_o, [lo], ones_v, mask=mo)
                dv = dbuf[j, pl.ds(g * 16, 16)]
                li = dv - nbase
                mi = (li >= 0) & (li < DEG_SH)
                plsc.addupdate_scatter(tab_i, [li], ones_v, mask=mi)
                return _

            return lax.fori_loop(0, 8, grp, None)

        return lax.fori_loop(0, BRD, row, None)

    fire(0, 0)

    def bloop(b, _):
        lax.cond(lax.rem(b, 2) == 0,
                 lambda x: block_slot(0, x),
                 lambda x: block_slot(1, x), b)
        return _

    lax.fori_loop(0, nblocks, bloop, None)
    pltpu.sync_copy(tab_o, out_hbm.at[eg, 0, pl.ds(nbase, DEG_SH)])
    pltpu.sync_copy(tab_i, out_hbm.at[eg, 1, pl.ds(nbase, DEG_SH)])


def _msg_body(srcp, dstp, h_hbm, agg_hbm,
              sb0, sb1, db0, db1, qs, qd, sia, sib, sic, dia, dib, dic,
              ra, rb, rc, zb, agg_sp,
              gsa, gsb, gsc, ssa, ssb, ssc, st0, st1):
    c = lax.axis_index("c")
    s = lax.axis_index("s")
    zv = jnp.zeros((16,), jnp.float32)

    def zb_init(i, _):
        zb[i // 6, pl.ds(lax.rem(i, 6) * 16, 16)] = zv
        return _

    lax.fori_loop(0, 64 * 6, zb_init, None)

    sibufs = (sia, sib, sic)
    dibufs = (dia, dib, dic)
    rows = (ra, rb, rc)
    gsems = (gsa, gsb, gsc)
    ssems = (ssa, ssb, ssc)
    sbufs = (sb0, sb1)
    dbufs = (db0, db1)
    stsems = (st0, st1)

    def flush_slot(p, fl, cnt):
        o = (p + 2) % 3

        @pl.when(fl >= 1)
        def _():
            pltpu.make_async_copy(h_hbm.at[sibufs[o]], rows[o],
                                  gsems[o]).wait()
            pltpu.async_copy(rows[o], agg_sp.at[dibufs[o]], ssems[o],
                             add=True)

        @pl.when(fl >= 3)
        def _():
            pltpu.make_async_copy(rows[p], agg_sp.at[dibufs[p]],
                                  ssems[p]).wait()

        for i in range(8):
            sibufs[p][pl.ds(i * 16, 16)] = qs[pl.ds(i * 16, 16)]
            dibufs[p][pl.ds(i * 16, 16)] = qd[pl.ds(i * 16, 16)]
        pltpu.async_copy(h_hbm.at[sibufs[p]], rows[p], gsems[p])
        for i in range(8):
            qs[pl.ds(i * 16, 16)] = qs[pl.ds(128 + i * 16, 16)]
            qd[pl.ds(i * 16, 16)] = qd[pl.ds(128 + i * 16, 16)]
        return cnt - 128, fl + 1

    def do_flush(state):
        cnt, fl = state
        r = lax.rem(fl, 3)
        return lax.cond(
            r == 0,
            lambda a, b: flush_slot(0, a, b),
            lambda a, b: lax.cond(a % 3 == 1,
                                  lambda x, y: flush_slot(1, x, y),
                                  lambda x, y: flush_slot(2, x, y), a, b),
            fl, cnt)

    def chunk_body(c7, _):
        @pl.when(lax.rem(c7, 2) == c)
        def _run():
            base = c7 * CH
            z0 = s * (CH // NS)
            for k in range(12):
                pltpu.sync_copy(zb, agg_sp.at[pl.ds(z0 + k * 64, 64)])
            pltpu.sync_copy(zb.at[pl.ds(0, 16)],
                            agg_sp.at[pl.ds(z0 + 768, 16)])
            plsc.subcore_barrier()

            nstages = EROWS_MSG // BR

            def fire_stage(bk, p):
                row0 = s * EROWS_MSG + bk * BR
                pltpu.async_copy(srcp.at[pl.ds(row0, BR)], sbufs[p],
                                 stsems[p])
                pltpu.async_copy(dstp.at[pl.ds(row0, BR)], dbufs[p],
                                 stsems[p])

            def stage_slot(sp, bk, state):
                row0 = s * EROWS_MSG + bk * BR
                pltpu.make_async_copy(srcp.at[pl.ds(row0, BR)], sbufs[sp],
                                      stsems[sp]).wait()
                pltpu.make_async_copy(dstp.at[pl.ds(row0, BR)], dbufs[sp],
                                      stsems[sp]).wait()

                @pl.when(bk + 1 < nstages)
                def _():
                    fire_stage(bk + 1, 1 - sp)

                sbuf = sbufs[sp]
                dbuf = dbufs[sp]

                def row(j, state):
                    def grp(g, cnt):
                        sv = sbuf[j, pl.ds(g * 16, 16)]
                        dv = dbuf[j, pl.ds(g * 16, 16)]
                        m = (dv >= base) & (dv < base + CH)
                        dl = dv - base
                        plsc.store_compressed(qs.at[pl.ds(cnt, 16)], sv,
                                              mask=m)
                        plsc.store_compressed(qd.at[pl.ds(cnt, 16)], dl,
                                              mask=m)
                        return cnt + plsc.all_reduce_population_count(m)[0]

                    cnt, fl = state
                    cnt = lax.fori_loop(0, 8, grp, cnt)
                    return lax.cond(cnt >= 128, do_flush,
                                    lambda x: x, (cnt, fl))

                return lax.fori_loop(0, BR, row, state)

            def stage(bk, state):
                return lax.cond(lax.rem(bk, 2) == 0,
                                lambda b, st: stage_slot(0, b, st),
                                lambda b, st: stage_slot(1, b, st),
                                bk, state)

            fire_stage(0, 0)
            state = lax.fori_loop(0, EROWS_MSG // BR, stage,
                                  (jnp.int32(0), jnp.int32(0)))
            cnt, fl = state
            pad_s = jnp.zeros((16,), jnp.int32)
            pad_d = jnp.full((16,), CH, jnp.int32)
            for i in range(8):
                qs[pl.ds(cnt + i * 16, 16)] = pad_s
                qd[pl.ds(cnt + i * 16, 16)] = pad_d
            _, fl = do_flush((cnt, fl))

            def drain_last(p):
                pltpu.make_async_copy(h_hbm.at[sibufs[p]], rows[p],
                                      gsems[p]).wait()
                pltpu.async_copy(rows[p], agg_sp.at[dibufs[p]], ssems[p],
                                 add=True)
                pltpu.make_async_copy(rows[p], agg_sp.at[dibufs[p]],
                                      ssems[p]).wait()

            def drain_prev(p):
                pltpu.make_async_copy(rows[p], agg_sp.at[dibufs[p]],
                                      ssems[p]).wait()

            def pick(r, fn):
                lax.cond(r == 0, lambda: fn(0),
                         lambda: lax.cond(r == 1, lambda: fn(1),
                                          lambda: fn(2)))

            pick(lax.rem(fl + 2, 3), drain_last)

            @pl.when(fl >= 2)
            def _():
                pick(lax.rem(fl + 1, 3), drain_prev)

            @pl.when(fl >= 3)
            def _():
                pick(lax.rem(fl, 3), drain_prev)

            plsc.subcore_barrier()
            for k in range(6):
                r = z0 + k * 128
                pltpu.sync_copy(agg_sp.at[pl.ds(r, 128)],
                                agg_hbm.at[pl.ds(base + r, 128)])
            pltpu.sync_copy(agg_sp.at[pl.ds(z0 + 768, 16)],
                            agg_hbm.at[pl.ds(base + z0 + 768, 16)])
            plsc.subcore_barrier()

        return _

    lax.fori_loop(0, NCHUNK, chunk_body, None)


_deg_call = functools.partial(
    pl.kernel, _deg_body,
    out_type=jax.ShapeDtypeStruct((EGRP, 2, NPH), jnp.float32),
    mesh=_mesh,
    compiler_params=_sc_params,
    scratch_types=[
        pltpu.VMEM((BRD, 128), jnp.int32),
        pltpu.VMEM((BRD, 128), jnp.int32),
        pltpu.VMEM((BRD, 128), jnp.int32),
        pltpu.VMEM((BRD, 128), jnp.int32),
        pltpu.VMEM((DEG_SH,), jnp.float32),
        pltpu.VMEM((DEG_SH,), jnp.float32),
        pltpu.SemaphoreType.DMA,
        pltpu.SemaphoreType.DMA,
    ],
)()

_msg_call = functools.partial(
    pl.kernel, _msg_body,
    out_type=jax.ShapeDtypeStruct((AGGR, DF), jnp.float32),
    mesh=_mesh,
    compiler_params=_sc_params,
    scratch_types=[
        pltpu.VMEM((BR, 128), jnp.int32),
        pltpu.VMEM((BR, 128), jnp.int32),
        pltpu.VMEM((BR, 128), jnp.int32),
        pltpu.VMEM((BR, 128), jnp.int32),
        pltpu.VMEM((QCAP,), jnp.int32),
        pltpu.VMEM((QCAP,), jnp.int32),
        pltpu.VMEM((128,), jnp.int32),
        pltpu.VMEM((128,), jnp.int32),
        pltpu.VMEM((128,), jnp.int32),
        pltpu.VMEM((128,), jnp.int32),
        pltpu.VMEM((128,), jnp.int32),
        pltpu.VMEM((128,), jnp.int32),
        pltpu.VMEM((128, DF), jnp.float32),
        pltpu.VMEM((128, DF), jnp.float32),
        pltpu.VMEM((128, DF), jnp.float32),
        pltpu.VMEM((64, DF), jnp.float32),
        pltpu.VMEM_SHARED((CH2, DF), jnp.float32),
        pltpu.SemaphoreType.DMA,
        pltpu.SemaphoreType.DMA,
        pltpu.SemaphoreType.DMA,
        pltpu.SemaphoreType.DMA,
        pltpu.SemaphoreType.DMA,
        pltpu.SemaphoreType.DMA,
        pltpu.SemaphoreType.DMA,
        pltpu.SemaphoreType.DMA,
    ],
)()


def _prep_body(feat_ref, *refs):
    deg_refs = refs[:2 * EGRP]
    h_ref, nd_ref = refs[2 * EGRP:]
    x0 = feat_ref[0]
    x1 = feat_ref[1]
    do = deg_refs[0][...]
    di = deg_refs[EGRP][...]
    for e in range(1, EGRP):
        do = do + deg_refs[e][...]
        di = di + deg_refs[EGRP + e][...]
    ns = jnp.where(do > 0, lax.rsqrt(jnp.maximum(do, 1.0)), 0.0)
    nd = jnp.where(di > 0, lax.rsqrt(jnp.maximum(di, 1.0)), 0.0)
    h_ref[:, 0:D_IN] = x0 * ns[:, None]
    h_ref[:, D_IN:DF] = x1 * ns[:, None]
    nd_ref[...] = nd


def _read_body(agg_ref, nd_ref, wc, bc, w0, b0, w1, b1, w2, b2, out_ref):
    nd = nd_ref[...][:, None]

    def mlp(a):
        y = jnp.maximum(jnp.dot(a, wc[...], preferred_element_type=jnp.float32) + bc[...], 0.0)
        y = jnp.maximum(jnp.dot(y, w0[...], preferred_element_type=jnp.float32) + b0[...], 0.0)
        y = jnp.maximum(jnp.dot(y, w1[...], preferred_element_type=jnp.float32) + b1[...], 0.0)
        return jnp.dot(y, w2[...], preferred_element_type=jnp.float32) + b2[...]

    out_ref[0] = mlp(agg_ref[:, 0:D_IN] * nd)
    out_ref[1] = mlp(agg_ref[:, D_IN:DF] * nd)


def _full(shape):
    return pl.BlockSpec(shape, lambda i: tuple(0 for _ in shape))


_prep_call = pl.pallas_call(
    _prep_body,
    grid=(NBLK,),
    in_specs=[pl.BlockSpec((B, BN, D_IN), lambda i: (0, i, 0))]
    + [pl.BlockSpec((BN,), lambda i: (i,)) for _ in range(2 * EGRP)],
    out_specs=[
        pl.BlockSpec((BN, DF), lambda i: (i, 0)),
        pl.BlockSpec((BN,), lambda i: (i,)),
    ],
    out_shape=[
        jax.ShapeDtypeStruct((NPH, DF), jnp.float32),
        jax.ShapeDtypeStruct((NPH,), jnp.float32),
    ],
)

_read_call = pl.pallas_call(
    _read_body,
    grid=(NBLK,),
    in_specs=[
        pl.BlockSpec((BN, DF), lambda i: (i, 0)),
        pl.BlockSpec((BN,), lambda i: (i,)),
        _full((D_IN, D_H)),
        _full((D_H,)),
        _full((D_H, D_L0)),
        _full((D_L0,)),
        _full((D_L0, D_H)),
        _full((D_H,)),
        _full((D_H, D_OUT)),
        _full((D_OUT,)),
    ],
    out_specs=pl.BlockSpec((B, BN, D_OUT), lambda i: (0, i, 0)),
    out_shape=jax.ShapeDtypeStruct((B, N, D_OUT), jnp.float32),
)


def kernel(feat, edge_index, W_conv, b_conv, W0, b0, W1, b1, W2, b2):
    src = edge_index[0]
    dst = edge_index[1]
    pad = N + 1 + (jnp.arange(EPAD, dtype=jnp.int32) % 300)
    srcp = jnp.concatenate([src, pad]).reshape(ER, 128)
    dstp = jnp.concatenate([dst, pad]).reshape(ER, 128)
    xf = feat.reshape(B, N, D_IN)
    featp = jnp.pad(xf, ((0, 0), (0, NPH - N), (0, 0)))

    degs = _deg_call(srcp, dstp)
    deg_parts = ([degs[e, 0] for e in range(EGRP)]
                 + [degs[e, 1] for e in range(EGRP)])
    h, nd = _prep_call(featp, *deg_parts)
    agg = _msg_call(srcp, dstp, h)
    out = _read_call(agg, nd, W_conv, b_conv, W0, b0, W1, b1, W2, b2)
    return out.reshape(B, N, 6, 4)
